# baseline (speedup 1.0000x reference)
import jax
import jax.numpy as jnp
from jax import lax
from jax.experimental import pallas as pl
from jax.experimental.pallas import tpu as pltpu
from jax.experimental.pallas import tpu_sc as plsc

NROWS = 50000
D = 128
E = 600000
NC, NS, L = 2, 16, 16

DB = 6400
NBLK = 8
NBPC = 4
TRASH = 128
ACC_ROWS = DB + TRASH

MACRO = 2048
SBATCH = 128
RBATCH = 256
EPT = 19 * MACRO
EP = NS * EPT
SEGCAP = EPT + MACRO

CW = 1024


def _zero_rows64(rowsbuf, zero16):
    for r in range(64):
        for j in range(D // L):
            rowsbuf[r, pl.ds(j * L, L)] = zero16


def _zero_acc(s, acc, rowsbuf):
    zrows = ACC_ROWS // NS
    nz = (zrows + 63) // 64
    for k in range(nz):
        r0 = s * zrows + k * 64
        r0 = pl.multiple_of(jnp.minimum(r0, ACC_ROWS - 64), 8)
        pltpu.sync_copy(rowsbuf.at[pl.ds(0, 64)], acc.at[pl.ds(r0, 64)])


def _own_rows(s, block):
    rows_real = jnp.where(block == NBLK - 1, NROWS - (NBLK - 1) * DB, DB)
    total16 = rows_real // 16
    n16 = (total16 + NS - 1) // NS
    start16 = s * n16
    mych = jnp.clip(total16 - start16, 0, n16)
    start_row = pl.multiple_of(start16 * 16, 16)
    return mych, start_row


def _scale_rows(q, cnt, rowsbuf):
    def scale(r, _2):
        iv = cnt[pl.ds(q * 16 + r, L)][0]
        ivv = jnp.full((L,), iv, jnp.float32)
        for j in range(D // L):
            rowsbuf[r, pl.ds(j * L, L)] = rowsbuf[r, pl.ds(j * L, L)] * ivv
        return 0
    lax.fori_loop(0, 16, scale, 0)


def _scan_body(table, srcp, dstp, out, srcc, dlocc, counts, invout,
               srcbuf, dstbuf, tmps, tmpd, sidx_send, dloc_send,
               sidx2_send, dloc2_send, rowsbuf, rows2buf,
               cnt, cslice, cntw, acc, cntstage,
               sem, sem2, ssem, ssem2, wsem, wsem2):
    c = lax.axis_index("c")
    s = lax.axis_index("s")
    slots = [(sidx_send, dloc_send, rowsbuf, sem, ssem),
             (sidx2_send, dloc2_send, rows2buf, sem2, ssem2)]
    zero16 = jnp.zeros((L,), jnp.float32)
    ones16 = jnp.ones((L,), jnp.float32)
    iota16 = lax.iota(jnp.int32, L)

    for b in range(NBPC):
        block = NBPC * c + b
        base = block * DB
        seg = pl.multiple_of((block * NS + s) * SEGCAP, 128)

        _zero_rows64(rowsbuf, zero16)

        def zero_cnt(i, _):
            cnt[pl.ds(i * L, L)] = zero16
            return 0
        lax.fori_loop(0, DB // L, zero_cnt, 0)

        def zero_tmps(i, _):
            tmps[pl.ds(i * L, L)] = jnp.zeros((L,), jnp.int32)
            return 0
        lax.fori_loop(0, (MACRO + SBATCH) // L, zero_tmps, 0)

        _zero_acc(s, acc, rowsbuf)
        plsc.subcore_barrier()

        def stage(kofs, sidx_d, dloc_d, fix_p):
            for j in range(SBATCH // L):
                sv = tmps[pl.ds(kofs + j * L, L)]
                dv = tmpd[pl.ds(kofs + j * L, L)]
                if fix_p is not None:
                    lane = j * L + iota16
                    keep = lane < fix_p
                    dv = jnp.where(keep, dv, DB + (lane & 127))
                sidx_d[pl.ds(j * L, L)] = sv
                dloc_d[pl.ds(j * L, L)] = dv

        def flush(kofs, fix_p):
            stage(kofs, sidx_send, dloc_send, fix_p)
            pltpu.async_copy(table.at[sidx_send], rowsbuf, sem).wait()
            pltpu.sync_copy(rowsbuf, acc.at[dloc_send], add=True)

        def macro_step(mi, carry):
            p, wofs = carry
            mbase = pl.multiple_of(s * EPT + mi * MACRO, MACRO)
            pltpu.sync_copy(srcp.at[pl.ds(mbase, MACRO)], srcbuf)
            pltpu.sync_copy(dstp.at[pl.ds(mbase, MACRO)], dstbuf)

            def compact(j, pp):
                d = dstbuf[pl.ds(j * L, L)]
                sv = srcbuf[pl.ds(j * L, L)]
                t = d - base
                inb = (t >= 0) & (t < DB)
                tc = jnp.where(inb, t, 0)
                plsc.addupdate_scatter(cnt, [tc], ones16, mask=inb)
                plsc.store_compressed(tmpd.at[pl.ds(pp, L)], t, mask=inb)
                plsc.store_compressed(tmps.at[pl.ds(pp, L)], sv, mask=inb)
                return pp + jnp.sum(inb.astype(jnp.int32))

            navail = lax.fori_loop(0, MACRO // L, compact, p)
            nb = navail // SBATCH

            aofs = pl.multiple_of(seg + wofs * SBATCH, 8)
            w1 = pltpu.async_copy(tmps.at[pl.ds(0, MACRO)],
                                  srcc.at[pl.ds(aofs, MACRO)], wsem)
            w2 = pltpu.async_copy(tmpd.at[pl.ds(0, MACRO)],
                                  dlocc.at[pl.ds(aofs, MACRO)], wsem2)

            def flush_n(kbase, nslots):
                gs = []
                for i in range(nslots):
                    sd, dd, rb, gsm, ssm = slots[i]
                    stage(kbase + i * SBATCH, sd, dd, None)
                    gs.append(pltpu.async_copy(table.at[sd], rb, gsm))
                ss = []
                for i in range(nslots):
                    sd, dd, rb, gsm, ssm = slots[i]
                    gs[i].wait()
                    ss.append(pltpu.async_copy(rb, acc.at[dd], ssm, add=True))
                for d_ in ss:
                    d_.wait()

            def flush_2(k, _):
                flush_n(k * 2 * SBATCH, 2)
                return 0
            lax.fori_loop(0, nb // 2, flush_2, 0)

            @pl.when(nb % 2 == 1)
            def _():
                flush((nb - 1) * SBATCH, None)

            w1.wait()
            w2.wait()

            rem = navail - nb * SBATCH

            @pl.when(nb > 0)
            def _():
                for t_ in range(SBATCH // L):
                    sv = tmps[pl.ds(nb * SBATCH + t_ * L, L)]
                    dv = tmpd[pl.ds(nb * SBATCH + t_ * L, L)]
                    tmps[pl.ds(t_ * L, L)] = sv
                    tmpd[pl.ds(t_ * L, L)] = dv
            return rem, wofs + nb

        p_final, wofs_final = lax.fori_loop(
            0, EPT // MACRO, macro_step, (jnp.int32(0), jnp.int32(0)))

        @pl.when(p_final > 0)
        def _():
            flush(0, p_final)

        n_tb = wofs_final * SBATCH + p_final
        cv = cntw[pl.ds(0, L)]
        cntw[pl.ds(0, L)] = jnp.where(iota16 == b, n_tb, cv)

        plsc.subcore_barrier()

        pltpu.sync_copy(cnt, cntstage.at[pl.ds(pl.multiple_of(s * DB, 128), DB)])
        plsc.subcore_barrier()

        mych, start_row = _own_rows(s, block)
        astart = pl.multiple_of(jnp.clip((start_row // 128) * 128, 0, DB - CW), 128)
        off = start_row - astart
        for r in range(NS):
            pltpu.sync_copy(cntstage.at[pl.ds(pl.multiple_of(r * DB + astart, 128), CW)],
                            cslice.at[pl.ds(r * CW, CW)])

        def merge(j, _):
            tot = cslice[pl.ds(off + j * L, L)]
            for r in range(1, NS):
                tot = tot + cslice[pl.ds(r * CW + off + j * L, L)]
            inv = 1.0 / jnp.maximum(tot, 1.0)
            cnt[pl.ds(j * L, L)] = inv
            return 0
        lax.fori_loop(0, mych, merge, 0)

        def out_chunk(q, _):
            r0 = pl.multiple_of(start_row + q * 16, 16)
            pltpu.sync_copy(acc.at[pl.ds(r0, 16)], rowsbuf.at[pl.ds(0, 16)])
            _scale_rows(q, cnt, rowsbuf)
            pltpu.sync_copy(rowsbuf.at[pl.ds(0, 16)],
                            out.at[pl.ds(base + r0, 16)])
            pltpu.sync_copy(cnt.at[pl.ds(pl.multiple_of(q * 16, 16), 16)],
                            invout.at[pl.ds(base + r0, 16)])
            return 0
        lax.fori_loop(0, mych, out_chunk, 0)
        plsc.subcore_barrier()

    pltpu.sync_copy(
        cntw, counts.at[pl.ds(pl.multiple_of((c * NS + s) * L, 16), L)])


def _replay_body(table, srcc, dlocc, counts, invin, out,
                 srcbuf, dstbuf, sidx_send, dloc_send,
                 sidx2_send, dloc2_send, rowsbuf, rows2buf,
                 cnt, cntw, acc,
                 sem, sem2, ssem, ssem2):
    c = lax.axis_index("c")
    s = lax.axis_index("s")
    slots = [(sidx_send, dloc_send, rowsbuf, sem, ssem),
             (sidx2_send, dloc2_send, rows2buf, sem2, ssem2)]
    zero16 = jnp.zeros((L,), jnp.float32)
    iota16 = lax.iota(jnp.int32, L)

    pltpu.sync_copy(
        counts.at[pl.ds(pl.multiple_of((c * NS + s) * L, 16), L)], cntw)

    for b in range(NBPC):
        block = NBPC * c + b
        base = block * DB
        seg = pl.multiple_of((block * NS + s) * SEGCAP, 128)

        _zero_rows64(rowsbuf, zero16)
        _zero_acc(s, acc, rowsbuf)
        plsc.subcore_barrier()

        n_tb = cntw[pl.ds(0, L)][b]
        nbat = (n_tb + RBATCH - 1) // RBATCH

        def stage_r(lofs, gofs, sidx_d, dloc_d):
            fp = n_tb - gofs * RBATCH
            for j in range(RBATCH // L):
                sv = srcbuf[pl.ds(lofs * RBATCH + j * L, L)]
                dv = dstbuf[pl.ds(lofs * RBATCH + j * L, L)]
                lane = j * L + iota16
                dv = jnp.where(lane < fp, dv, DB + (lane & 127))
                sidx_d[pl.ds(j * L, L)] = sv
                dloc_d[pl.ds(j * L, L)] = dv

        def chunk_step(mi, _):
            cofs = pl.multiple_of(seg + mi * MACRO, 8)
            pltpu.sync_copy(srcc.at[pl.ds(cofs, MACRO)], srcbuf)
            pltpu.sync_copy(dlocc.at[pl.ds(cofs, MACRO)], dstbuf)
            nb_c = jnp.minimum(nbat - mi * (MACRO // RBATCH), MACRO // RBATCH)

            def flush_n(lbase, nslots):
                gs = []
                for i in range(nslots):
                    sd, dd, rb, gsm, ssm = slots[i]
                    stage_r(lbase + i, mi * (MACRO // RBATCH) + lbase + i, sd, dd)
                    gs.append(pltpu.async_copy(table.at[sd], rb, gsm))
                ss = []
                for i in range(nslots):
                    sd, dd, rb, gsm, ssm = slots[i]
                    gs[i].wait()
                    ss.append(pltpu.async_copy(rb, acc.at[dd], ssm, add=True))
                for d_ in ss:
                    d_.wait()

            def flush_2(k, _2):
                flush_n(2 * k, 2)
                return 0
            lax.fori_loop(0, nb_c // 2, flush_2, 0)

            @pl.when(nb_c % 2 == 1)
            def _():
                flush_n(nb_c - 1, 1)
            return 0
        nch = (nbat + (MACRO // RBATCH) - 1) // (MACRO // RBATCH)
        lax.fori_loop(0, nch, chunk_step, 0)
        plsc.subcore_barrier()

        mych, start_row = _own_rows(s, block)

        def out_chunk(q, _):
            r0 = pl.multiple_of(start_row + q * 16, 16)
            pltpu.sync_copy(acc.at[pl.ds(r0, 16)], rowsbuf.at[pl.ds(0, 16)])
            pltpu.sync_copy(invin.at[pl.ds(base + r0, 16)],
                            cnt.at[pl.ds(pl.multiple_of(q * 16, 16), 16)])
            _scale_rows(q, cnt, rowsbuf)
            pltpu.sync_copy(rowsbuf.at[pl.ds(0, 16)],
                            out.at[pl.ds(base + r0, 16)])
            return 0
        lax.fori_loop(0, mych, out_chunk, 0)
        plsc.subcore_barrier()


_MESH = plsc.VectorSubcoreMesh(core_axis_name="c", subcore_axis_name="s",
                               num_cores=NC, num_subcores=NS)
_SEGTOT = NBLK * NS * SEGCAP


@jax.jit
def _smean_scan(table, srcp, dstp):
    f = pl.kernel(
        _scan_body,
        out_type=(
            jax.ShapeDtypeStruct((NROWS, D), jnp.float32),
            jax.ShapeDtypeStruct((_SEGTOT,), jnp.int32),
            jax.ShapeDtypeStruct((_SEGTOT,), jnp.int32),
            jax.ShapeDtypeStruct((NC * NS * L,), jnp.int32),
            jax.ShapeDtypeStruct((NROWS,), jnp.float32),
        ),
        mesh=_MESH,
        scratch_types=[
            pltpu.VMEM((MACRO,), jnp.int32),
            pltpu.VMEM((MACRO,), jnp.int32),
            pltpu.VMEM((MACRO + SBATCH,), jnp.int32),
            pltpu.VMEM((MACRO + SBATCH,), jnp.int32),
            pltpu.VMEM((SBATCH,), jnp.int32),
            pltpu.VMEM((SBATCH,), jnp.int32),
            pltpu.VMEM((SBATCH,), jnp.int32),
            pltpu.VMEM((SBATCH,), jnp.int32),
            pltpu.VMEM((SBATCH, D), jnp.float32),
            pltpu.VMEM((SBATCH, D), jnp.float32),
            pltpu.VMEM((DB,), jnp.float32),
            pltpu.VMEM((NS * CW,), jnp.float32),
            pltpu.VMEM((L,), jnp.int32),
            pltpu.VMEM_SHARED((ACC_ROWS, D), jnp.float32),
            pltpu.VMEM_SHARED((NS * DB,), jnp.float32),
        ] + [pltpu.SemaphoreType.DMA] * 6,
        compiler_params=pltpu.CompilerParams(needs_layout_passes=False),
    )
    return f(table, srcp, dstp)


@jax.jit
def _smean_replay(table, srcc, dlocc, counts, invin):
    f = pl.kernel(
        _replay_body,
        out_type=jax.ShapeDtypeStruct((NROWS, D), jnp.float32),
        mesh=_MESH,
        scratch_types=[
            pltpu.VMEM((MACRO,), jnp.int32),
            pltpu.VMEM((MACRO,), jnp.int32),
            pltpu.VMEM((RBATCH,), jnp.int32),
            pltpu.VMEM((RBATCH,), jnp.int32),
            pltpu.VMEM((RBATCH,), jnp.int32),
            pltpu.VMEM((RBATCH,), jnp.int32),
            pltpu.VMEM((RBATCH, D), jnp.float32),
            pltpu.VMEM((RBATCH, D), jnp.float32),
            pltpu.VMEM((DB,), jnp.float32),
            pltpu.VMEM((L,), jnp.int32),
            pltpu.VMEM_SHARED((ACC_ROWS, D), jnp.float32),
        ] + [pltpu.SemaphoreType.DMA] * 4,
        compiler_params=pltpu.CompilerParams(needs_layout_passes=False),
    )
    return f(table, srcc, dlocc, counts, invin)


def _pad_edges(e):
    src = e[0].astype(jnp.int32)
    dst = e[1].astype(jnp.int32)
    pad = EP - E
    srcp = jnp.concatenate([src, jnp.zeros((pad,), jnp.int32)])
    dstp = jnp.concatenate([dst, jnp.full((pad,), -1, jnp.int32)])
    return srcp, dstp


def kernel(x_users, x_artists, edge_index_a2u, edge_index_u2a):
    sa, da = _pad_edges(edge_index_a2u)
    su, du = _pad_edges(edge_index_u2a)
    xu, xa = x_users, x_artists
    xu, a_srcc, a_dlocc, a_counts, a_inv = _smean_scan(xa, sa, da)
    xa, u_srcc, u_dlocc, u_counts, u_inv = _smean_scan(xu, su, du)
    fu = x_users + xu
    fa = x_artists + xa
    for _ in range(2):
        xu = _smean_replay(xa, a_srcc, a_dlocc, a_counts, a_inv)
        xa = _smean_replay(xu, u_srcc, u_dlocc, u_counts, u_inv)
        fu = fu + xu
        fa = fa + xa
    return (0.25 * fu, 0.25 * fa)


# speedup vs baseline: 1.3628x; 1.3628x over previous
import jax
import jax.numpy as jnp
from jax import lax
from jax.experimental import pallas as pl
from jax.experimental.pallas import tpu as pltpu
from jax.experimental.pallas import tpu_sc as plsc

NROWS = 50000
D = 128
E = 600000
NC, NS, L = 2, 16, 16

DB = 6400
NBLK = 8
NBPC = 4
TRASH = 128
ACC_ROWS = DB + TRASH

MACRO = 2048
SBATCH = 128
RBATCH = 128
EPT = 19 * MACRO
EP = NS * EPT
SEGCAP = EPT + MACRO

CW = 1024


def _zero_rows64(rowsbuf, zero16):
    for r in range(64):
        for j in range(D // L):
            rowsbuf[r, pl.ds(j * L, L)] = zero16


def _zero_acc(s, acc, rowsbuf):
    zrows = ACC_ROWS // NS
    nz = (zrows + 63) // 64
    for k in range(nz):
        r0 = s * zrows + k * 64
        r0 = pl.multiple_of(jnp.minimum(r0, ACC_ROWS - 64), 8)
        pltpu.sync_copy(rowsbuf.at[pl.ds(0, 64)], acc.at[pl.ds(r0, 64)])


def _own_rows(s, block):
    rows_real = jnp.where(block == NBLK - 1, NROWS - (NBLK - 1) * DB, DB)
    total16 = rows_real // 16
    n16 = (total16 + NS - 1) // NS
    start16 = s * n16
    mych = jnp.clip(total16 - start16, 0, n16)
    start_row = pl.multiple_of(start16 * 16, 16)
    return mych, start_row


def _scale_rows(q, cnt, rowsbuf):
    def scale(r, _2):
        iv = cnt[pl.ds(q * 16 + r, L)][0]
        ivv = jnp.full((L,), iv, jnp.float32)
        for j in range(D // L):
            rowsbuf[r, pl.ds(j * L, L)] = rowsbuf[r, pl.ds(j * L, L)] * ivv
        return 0
    lax.fori_loop(0, 16, scale, 0)


def _scan_body(table, srcp, dstp, out, srcc, dlocc, counts, invout,
               srcbuf, dstbuf, tmps, tmpd, sidx_send, dloc_send,
               sidx2_send, dloc2_send, rowsbuf, rows2buf,
               cnt, cslice, cntw, acc, cntstage,
               sem, sem2, ssem, ssem2, wsem, wsem2):
    c = lax.axis_index("c")
    s = lax.axis_index("s")
    slots = [(sidx_send, dloc_send, rowsbuf, sem, ssem),
             (sidx2_send, dloc2_send, rows2buf, sem2, ssem2)]
    zero16 = jnp.zeros((L,), jnp.float32)
    ones16 = jnp.ones((L,), jnp.float32)
    iota16 = lax.iota(jnp.int32, L)

    for b in range(NBPC):
        block = NBPC * c + b
        base = block * DB
        seg = pl.multiple_of((block * NS + s) * SEGCAP, 128)

        _zero_rows64(rowsbuf, zero16)

        def zero_cnt(i, _):
            cnt[pl.ds(i * L, L)] = zero16
            return 0
        lax.fori_loop(0, DB // L, zero_cnt, 0)

        def zero_tmps(i, _):
            tmps[pl.ds(i * L, L)] = jnp.zeros((L,), jnp.int32)
            return 0
        lax.fori_loop(0, (MACRO + SBATCH) // L, zero_tmps, 0)

        _zero_acc(s, acc, rowsbuf)
        plsc.subcore_barrier()

        def stage(kofs, sidx_d, dloc_d, fix_p):
            for j in range(SBATCH // L):
                sv = tmps[pl.ds(kofs + j * L, L)]
                dv = tmpd[pl.ds(kofs + j * L, L)]
                if fix_p is not None:
                    lane = j * L + iota16
                    keep = lane < fix_p
                    dv = jnp.where(keep, dv, DB + (lane & 127))
                sidx_d[pl.ds(j * L, L)] = sv
                dloc_d[pl.ds(j * L, L)] = dv

        def flush(kofs, fix_p):
            stage(kofs, sidx_send, dloc_send, fix_p)
            pltpu.async_copy(table.at[sidx_send], rowsbuf, sem).wait()
            pltpu.sync_copy(rowsbuf, acc.at[dloc_send], add=True)

        def macro_step(mi, carry):
            p, wofs = carry
            mbase = pl.multiple_of(s * EPT + mi * MACRO, MACRO)
            pltpu.sync_copy(srcp.at[pl.ds(mbase, MACRO)], srcbuf)
            pltpu.sync_copy(dstp.at[pl.ds(mbase, MACRO)], dstbuf)

            def compact(j, pp):
                d = dstbuf[pl.ds(j * L, L)]
                sv = srcbuf[pl.ds(j * L, L)]
                t = d - base
                inb = (t >= 0) & (t < DB)
                tc = jnp.where(inb, t, 0)
                plsc.addupdate_scatter(cnt, [tc], ones16, mask=inb)
                plsc.store_compressed(tmpd.at[pl.ds(pp, L)], t, mask=inb)
                plsc.store_compressed(tmps.at[pl.ds(pp, L)], sv, mask=inb)
                return pp + jnp.sum(inb.astype(jnp.int32))

            navail = lax.fori_loop(0, MACRO // L, compact, p)
            nb = navail // SBATCH

            aofs = pl.multiple_of(seg + wofs * SBATCH, 8)
            w1 = pltpu.async_copy(tmps.at[pl.ds(0, MACRO)],
                                  srcc.at[pl.ds(aofs, MACRO)], wsem)
            w2 = pltpu.async_copy(tmpd.at[pl.ds(0, MACRO)],
                                  dlocc.at[pl.ds(aofs, MACRO)], wsem2)

            def flush_n(kbase, nslots):
                gs = []
                for i in range(nslots):
                    sd, dd, rb, gsm, ssm = slots[i]
                    stage(kbase + i * SBATCH, sd, dd, None)
                    gs.append(pltpu.async_copy(table.at[sd], rb, gsm))
                ss = []
                for i in range(nslots):
                    sd, dd, rb, gsm, ssm = slots[i]
                    gs[i].wait()
                    ss.append(pltpu.async_copy(rb, acc.at[dd], ssm, add=True))
                for d_ in ss:
                    d_.wait()

            def flush_2(k, _):
                flush_n(k * 2 * SBATCH, 2)
                return 0
            lax.fori_loop(0, nb // 2, flush_2, 0)

            @pl.when(nb % 2 == 1)
            def _():
                flush((nb - 1) * SBATCH, None)

            w1.wait()
            w2.wait()

            rem = navail - nb * SBATCH

            @pl.when(nb > 0)
            def _():
                for t_ in range(SBATCH // L):
                    sv = tmps[pl.ds(nb * SBATCH + t_ * L, L)]
                    dv = tmpd[pl.ds(nb * SBATCH + t_ * L, L)]
                    tmps[pl.ds(t_ * L, L)] = sv
                    tmpd[pl.ds(t_ * L, L)] = dv
            return rem, wofs + nb

        p_final, wofs_final = lax.fori_loop(
            0, EPT // MACRO, macro_step, (jnp.int32(0), jnp.int32(0)))

        @pl.when(p_final > 0)
        def _():
            flush(0, p_final)

        n_tb = wofs_final * SBATCH + p_final
        cv = cntw[pl.ds(0, L)]
        cntw[pl.ds(0, L)] = jnp.where(iota16 == b, n_tb, cv)

        plsc.subcore_barrier()

        pltpu.sync_copy(cnt, cntstage.at[pl.ds(pl.multiple_of(s * DB, 128), DB)])
        plsc.subcore_barrier()

        mych, start_row = _own_rows(s, block)
        astart = pl.multiple_of(jnp.clip((start_row // 128) * 128, 0, DB - CW), 128)
        off = start_row - astart
        for r in range(NS):
            pltpu.sync_copy(cntstage.at[pl.ds(pl.multiple_of(r * DB + astart, 128), CW)],
                            cslice.at[pl.ds(r * CW, CW)])

        def merge(j, _):
            tot = cslice[pl.ds(off + j * L, L)]
            for r in range(1, NS):
                tot = tot + cslice[pl.ds(r * CW + off + j * L, L)]
            inv = 1.0 / jnp.maximum(tot, 1.0)
            cnt[pl.ds(j * L, L)] = inv
            return 0
        lax.fori_loop(0, mych, merge, 0)

        def out_chunk(q, _):
            r0 = pl.multiple_of(start_row + q * 16, 16)
            pltpu.sync_copy(acc.at[pl.ds(r0, 16)], rowsbuf.at[pl.ds(0, 16)])
            _scale_rows(q, cnt, rowsbuf)
            pltpu.sync_copy(rowsbuf.at[pl.ds(0, 16)],
                            out.at[pl.ds(base + r0, 16)])
            pltpu.sync_copy(cnt.at[pl.ds(pl.multiple_of(q * 16, 16), 16)],
                            invout.at[pl.ds(base + r0, 16)])
            return 0
        lax.fori_loop(0, mych, out_chunk, 0)
        plsc.subcore_barrier()

    pltpu.sync_copy(
        cntw, counts.at[pl.ds(pl.multiple_of((c * NS + s) * L, 16), L)])


def _replay_body(table, srcc, dlocc, counts, invin, out,
                 srcbuf, dstbuf, sidx_send, dloc_send,
                 sidx2_send, dloc2_send, sidx3_send, dloc3_send,
                 sidx4_send, dloc4_send, rowsbuf, rows2buf, rows3buf, rows4buf,
                 cnt, cntw, acc,
                 sem, sem2, sem3, sem4, ssem, ssem2, ssem3, ssem4):
    c = lax.axis_index("c")
    s = lax.axis_index("s")
    slots = [(sidx_send, dloc_send, rowsbuf, sem, ssem),
             (sidx2_send, dloc2_send, rows2buf, sem2, ssem2),
             (sidx3_send, dloc3_send, rows3buf, sem3, ssem3),
             (sidx4_send, dloc4_send, rows4buf, sem4, ssem4)]
    zero16 = jnp.zeros((L,), jnp.float32)
    iota16 = lax.iota(jnp.int32, L)

    pltpu.sync_copy(
        counts.at[pl.ds(pl.multiple_of((c * NS + s) * L, 16), L)], cntw)

    for b in range(NBPC):
        block = NBPC * c + b
        base = block * DB
        seg = pl.multiple_of((block * NS + s) * SEGCAP, 128)

        _zero_rows64(rowsbuf, zero16)
        _zero_acc(s, acc, rowsbuf)
        plsc.subcore_barrier()

        n_tb = cntw[pl.ds(0, L)][b]
        nbat = (n_tb + RBATCH - 1) // RBATCH

        def stage_r(lofs, gofs, sidx_d, dloc_d):
            fp = n_tb - gofs * RBATCH
            for j in range(RBATCH // L):
                sv = srcbuf[pl.ds(lofs * RBATCH + j * L, L)]
                dv = dstbuf[pl.ds(lofs * RBATCH + j * L, L)]
                lane = j * L + iota16
                dv = jnp.where(lane < fp, dv, DB + (lane & 127))
                sidx_d[pl.ds(j * L, L)] = sv
                dloc_d[pl.ds(j * L, L)] = dv

        def chunk_step(mi, _):
            cofs = pl.multiple_of(seg + mi * MACRO, 8)
            pltpu.sync_copy(srcc.at[pl.ds(cofs, MACRO)], srcbuf)
            pltpu.sync_copy(dlocc.at[pl.ds(cofs, MACRO)], dstbuf)
            nb_c = jnp.minimum(nbat - mi * (MACRO // RBATCH), MACRO // RBATCH)

            def flush_n(lbase, nslots):
                gs = []
                for i in range(nslots):
                    sd, dd, rb, gsm, ssm = slots[i]
                    stage_r(lbase + i, mi * (MACRO // RBATCH) + lbase + i, sd, dd)
                    gs.append(pltpu.async_copy(table.at[sd], rb, gsm))
                ss = []
                for i in range(nslots):
                    sd, dd, rb, gsm, ssm = slots[i]
                    gs[i].wait()
                    ss.append(pltpu.async_copy(rb, acc.at[dd], ssm, add=True))
                for d_ in ss:
                    d_.wait()

            def flush_4(k, _2):
                flush_n(4 * k, 4)
                return 0
            lax.fori_loop(0, nb_c // 4, flush_4, 0)

            @pl.when(nb_c % 4 >= 2)
            def _():
                flush_n(nb_c - (nb_c % 4), 2)

            @pl.when(nb_c % 2 == 1)
            def _():
                flush_n(nb_c - 1, 1)
            return 0
        nch = (nbat + (MACRO // RBATCH) - 1) // (MACRO // RBATCH)
        lax.fori_loop(0, nch, chunk_step, 0)
        plsc.subcore_barrier()

        mych, start_row = _own_rows(s, block)

        def out_chunk(q, _):
            r0 = pl.multiple_of(start_row + q * 16, 16)
            pltpu.sync_copy(acc.at[pl.ds(r0, 16)], rowsbuf.at[pl.ds(0, 16)])
            pltpu.sync_copy(invin.at[pl.ds(base + r0, 16)],
                            cnt.at[pl.ds(pl.multiple_of(q * 16, 16), 16)])
            _scale_rows(q, cnt, rowsbuf)
            pltpu.sync_copy(rowsbuf.at[pl.ds(0, 16)],
                            out.at[pl.ds(base + r0, 16)])
            return 0
        lax.fori_loop(0, mych, out_chunk, 0)
        plsc.subcore_barrier()


_MESH = plsc.VectorSubcoreMesh(core_axis_name="c", subcore_axis_name="s",
                               num_cores=NC, num_subcores=NS)
_SEGTOT = NBLK * NS * SEGCAP


@jax.jit
def _smean_scan(table, srcp, dstp):
    f = pl.kernel(
        _scan_body,
        out_type=(
            jax.ShapeDtypeStruct((NROWS, D), jnp.float32),
            jax.ShapeDtypeStruct((_SEGTOT,), jnp.int32),
            jax.ShapeDtypeStruct((_SEGTOT,), jnp.int32),
            jax.ShapeDtypeStruct((NC * NS * L,), jnp.int32),
            jax.ShapeDtypeStruct((NROWS,), jnp.float32),
        ),
        mesh=_MESH,
        scratch_types=[
            pltpu.VMEM((MACRO,), jnp.int32),
            pltpu.VMEM((MACRO,), jnp.int32),
            pltpu.VMEM((MACRO + SBATCH,), jnp.int32),
            pltpu.VMEM((MACRO + SBATCH,), jnp.int32),
            pltpu.VMEM((SBATCH,), jnp.int32),
            pltpu.VMEM((SBATCH,), jnp.int32),
            pltpu.VMEM((SBATCH,), jnp.int32),
            pltpu.VMEM((SBATCH,), jnp.int32),
            pltpu.VMEM((SBATCH, D), jnp.float32),
            pltpu.VMEM((SBATCH, D), jnp.float32),
            pltpu.VMEM((DB,), jnp.float32),
            pltpu.VMEM((NS * CW,), jnp.float32),
            pltpu.VMEM((L,), jnp.int32),
            pltpu.VMEM_SHARED((ACC_ROWS, D), jnp.float32),
            pltpu.VMEM_SHARED((NS * DB,), jnp.float32),
        ] + [pltpu.SemaphoreType.DMA] * 6,
        compiler_params=pltpu.CompilerParams(needs_layout_passes=False),
    )
    return f(table, srcp, dstp)


@jax.jit
def _smean_replay(table, srcc, dlocc, counts, invin):
    f = pl.kernel(
        _replay_body,
        out_type=jax.ShapeDtypeStruct((NROWS, D), jnp.float32),
        mesh=_MESH,
        scratch_types=[
            pltpu.VMEM((MACRO,), jnp.int32),
            pltpu.VMEM((MACRO,), jnp.int32),
            pltpu.VMEM((RBATCH,), jnp.int32),
            pltpu.VMEM((RBATCH,), jnp.int32),
            pltpu.VMEM((RBATCH,), jnp.int32),
            pltpu.VMEM((RBATCH,), jnp.int32),
            pltpu.VMEM((RBATCH,), jnp.int32),
            pltpu.VMEM((RBATCH,), jnp.int32),
            pltpu.VMEM((RBATCH,), jnp.int32),
            pltpu.VMEM((RBATCH,), jnp.int32),
            pltpu.VMEM((RBATCH, D), jnp.float32),
            pltpu.VMEM((RBATCH, D), jnp.float32),
            pltpu.VMEM((RBATCH, D), jnp.float32),
            pltpu.VMEM((RBATCH, D), jnp.float32),
            pltpu.VMEM((DB,), jnp.float32),
            pltpu.VMEM((L,), jnp.int32),
            pltpu.VMEM_SHARED((ACC_ROWS, D), jnp.float32),
        ] + [pltpu.SemaphoreType.DMA] * 8,
        compiler_params=pltpu.CompilerParams(needs_layout_passes=False),
    )
    return f(table, srcc, dlocc, counts, invin)


def _pad_edges(e):
    src = e[0].astype(jnp.int32)
    dst = e[1].astype(jnp.int32)
    pad = EP - E
    srcp = jnp.concatenate([src, jnp.zeros((pad,), jnp.int32)])
    dstp = jnp.concatenate([dst, jnp.full((pad,), -1, jnp.int32)])
    return srcp, dstp


def kernel(x_users, x_artists, edge_index_a2u, edge_index_u2a):
    sa, da = _pad_edges(edge_index_a2u)
    su, du = _pad_edges(edge_index_u2a)
    xu, xa = x_users, x_artists
    xu, a_srcc, a_dlocc, a_counts, a_inv = _smean_scan(xa, sa, da)
    xa, u_srcc, u_dlocc, u_counts, u_inv = _smean_scan(xu, su, du)
    fu = x_users + xu
    fa = x_artists + xa
    for _ in range(2):
        xu = _smean_replay(xa, a_srcc, a_dlocc, a_counts, a_inv)
        xa = _smean_replay(xu, u_srcc, u_dlocc, u_counts, u_inv)
        fu = fu + xu
        fa = fa + xa
    return (0.25 * fu, 0.25 * fa)


# speedup vs baseline: 1.3654x; 1.0019x over previous
import jax
import jax.numpy as jnp
from jax import lax
from jax.experimental import pallas as pl
from jax.experimental.pallas import tpu as pltpu
from jax.experimental.pallas import tpu_sc as plsc

NROWS = 50000
D = 128
E = 600000
NC, NS, L = 2, 16, 16

DB = 6400
NBLK = 8
NBPC = 4
TRASH = 128
ACC_ROWS = DB + TRASH

MACRO = 2048
SBATCH = 128
RBATCH = 128
EPT = 19 * MACRO
EP = NS * EPT
SEGCAP = EPT + MACRO

CW = 1024


def _zero_rows64(rowsbuf, zero16):
    for r in range(64):
        for j in range(D // L):
            rowsbuf[r, pl.ds(j * L, L)] = zero16


def _zero_acc(s, acc, rowsbuf):
    zrows = ACC_ROWS // NS
    nz = (zrows + 63) // 64
    for k in range(nz):
        r0 = s * zrows + k * 64
        r0 = pl.multiple_of(jnp.minimum(r0, ACC_ROWS - 64), 8)
        pltpu.sync_copy(rowsbuf.at[pl.ds(0, 64)], acc.at[pl.ds(r0, 64)])


def _own_rows(s, block):
    rows_real = jnp.where(block == NBLK - 1, NROWS - (NBLK - 1) * DB, DB)
    total16 = rows_real // 16
    n16 = (total16 + NS - 1) // NS
    start16 = s * n16
    mych = jnp.clip(total16 - start16, 0, n16)
    start_row = pl.multiple_of(start16 * 16, 16)
    return mych, start_row


def _scale_rows(q, cnt, rowsbuf):
    def scale(r, _2):
        iv = cnt[pl.ds(q * 16 + r, L)][0]
        ivv = jnp.full((L,), iv, jnp.float32)
        for j in range(D // L):
            rowsbuf[r, pl.ds(j * L, L)] = rowsbuf[r, pl.ds(j * L, L)] * ivv
        return 0
    lax.fori_loop(0, 16, scale, 0)


def _scan_body(table, srcp, dstp, out, srcc, dlocc, counts, invout,
               srcbuf, dstbuf, tmps, tmpd, sidx_send, dloc_send,
               sidx2_send, dloc2_send, rowsbuf, rows2buf,
               cnt, cslice, cntw, acc, cntstage,
               sem, sem2, ssem, ssem2, wsem, wsem2):
    c = lax.axis_index("c")
    s = lax.axis_index("s")
    slots = [(sidx_send, dloc_send, rowsbuf, sem, ssem),
             (sidx2_send, dloc2_send, rows2buf, sem2, ssem2)]
    zero16 = jnp.zeros((L,), jnp.float32)
    ones16 = jnp.ones((L,), jnp.float32)
    iota16 = lax.iota(jnp.int32, L)

    for b in range(NBPC):
        block = NBPC * c + b
        base = block * DB
        seg = pl.multiple_of((block * NS + s) * SEGCAP, 128)

        _zero_rows64(rowsbuf, zero16)

        def zero_cnt(i, _):
            cnt[pl.ds(i * L, L)] = zero16
            return 0
        lax.fori_loop(0, DB // L, zero_cnt, 0)

        def zero_tmps(i, _):
            tmps[pl.ds(i * L, L)] = jnp.zeros((L,), jnp.int32)
            return 0
        lax.fori_loop(0, (MACRO + SBATCH) // L, zero_tmps, 0)

        _zero_acc(s, acc, rowsbuf)
        plsc.subcore_barrier()

        def stage(kofs, sidx_d, dloc_d, fix_p):
            for j in range(SBATCH // L):
                sv = tmps[pl.ds(kofs + j * L, L)]
                dv = tmpd[pl.ds(kofs + j * L, L)]
                if fix_p is not None:
                    lane = j * L + iota16
                    keep = lane < fix_p
                    dv = jnp.where(keep, dv, DB + (lane & 127))
                sidx_d[pl.ds(j * L, L)] = sv
                dloc_d[pl.ds(j * L, L)] = dv

        def flush(kofs, fix_p):
            stage(kofs, sidx_send, dloc_send, fix_p)
            pltpu.async_copy(table.at[sidx_send], rowsbuf, sem).wait()
            pltpu.sync_copy(rowsbuf, acc.at[dloc_send], add=True)

        def macro_step(mi, carry):
            p, wofs = carry
            mbase = pl.multiple_of(s * EPT + mi * MACRO, MACRO)
            pltpu.sync_copy(srcp.at[pl.ds(mbase, MACRO)], srcbuf)
            pltpu.sync_copy(dstp.at[pl.ds(mbase, MACRO)], dstbuf)

            def compact(j, pp):
                d = dstbuf[pl.ds(j * L, L)]
                sv = srcbuf[pl.ds(j * L, L)]
                t = d - base
                inb = (t >= 0) & (t < DB)
                tc = jnp.where(inb, t, 0)
                plsc.addupdate_scatter(cnt, [tc], ones16, mask=inb)
                plsc.store_compressed(tmpd.at[pl.ds(pp, L)], t, mask=inb)
                plsc.store_compressed(tmps.at[pl.ds(pp, L)], sv, mask=inb)
                return pp + plsc.all_reduce_population_count(inb)[0]

            navail = lax.fori_loop(0, MACRO // L, compact, p)
            nb = navail // SBATCH

            aofs = pl.multiple_of(seg + wofs * SBATCH, 8)
            w1 = pltpu.async_copy(tmps.at[pl.ds(0, MACRO)],
                                  srcc.at[pl.ds(aofs, MACRO)], wsem)
            w2 = pltpu.async_copy(tmpd.at[pl.ds(0, MACRO)],
                                  dlocc.at[pl.ds(aofs, MACRO)], wsem2)

            def flush_n(kbase, nslots):
                gs = []
                for i in range(nslots):
                    sd, dd, rb, gsm, ssm = slots[i]
                    stage(kbase + i * SBATCH, sd, dd, None)
                    gs.append(pltpu.async_copy(table.at[sd], rb, gsm))
                ss = []
                for i in range(nslots):
                    sd, dd, rb, gsm, ssm = slots[i]
                    gs[i].wait()
                    ss.append(pltpu.async_copy(rb, acc.at[dd], ssm, add=True))
                for d_ in ss:
                    d_.wait()

            def flush_2(k, _):
                flush_n(k * 2 * SBATCH, 2)
                return 0
            lax.fori_loop(0, nb // 2, flush_2, 0)

            @pl.when(nb % 2 == 1)
            def _():
                flush((nb - 1) * SBATCH, None)

            w1.wait()
            w2.wait()

            rem = navail - nb * SBATCH

            @pl.when(nb > 0)
            def _():
                for t_ in range(SBATCH // L):
                    sv = tmps[pl.ds(nb * SBATCH + t_ * L, L)]
                    dv = tmpd[pl.ds(nb * SBATCH + t_ * L, L)]
                    tmps[pl.ds(t_ * L, L)] = sv
                    tmpd[pl.ds(t_ * L, L)] = dv
            return rem, wofs + nb

        p_final, wofs_final = lax.fori_loop(
            0, EPT // MACRO, macro_step, (jnp.int32(0), jnp.int32(0)))

        @pl.when(p_final > 0)
        def _():
            flush(0, p_final)

        n_tb = wofs_final * SBATCH + p_final
        cv = cntw[pl.ds(0, L)]
        cntw[pl.ds(0, L)] = jnp.where(iota16 == b, n_tb, cv)

        plsc.subcore_barrier()

        pltpu.sync_copy(cnt, cntstage.at[pl.ds(pl.multiple_of(s * DB, 128), DB)])
        plsc.subcore_barrier()

        mych, start_row = _own_rows(s, block)
        astart = pl.multiple_of(jnp.clip((start_row // 128) * 128, 0, DB - CW), 128)
        off = start_row - astart
        for r in range(NS):
            pltpu.sync_copy(cntstage.at[pl.ds(pl.multiple_of(r * DB + astart, 128), CW)],
                            cslice.at[pl.ds(r * CW, CW)])

        def merge(j, _):
            tot = cslice[pl.ds(off + j * L, L)]
            for r in range(1, NS):
                tot = tot + cslice[pl.ds(r * CW + off + j * L, L)]
            inv = 1.0 / jnp.maximum(tot, 1.0)
            cnt[pl.ds(j * L, L)] = inv
            return 0
        lax.fori_loop(0, mych, merge, 0)

        def out_chunk(q, _):
            r0 = pl.multiple_of(start_row + q * 16, 16)
            pltpu.sync_copy(acc.at[pl.ds(r0, 16)], rowsbuf.at[pl.ds(0, 16)])
            _scale_rows(q, cnt, rowsbuf)
            pltpu.sync_copy(rowsbuf.at[pl.ds(0, 16)],
                            out.at[pl.ds(base + r0, 16)])
            pltpu.sync_copy(cnt.at[pl.ds(pl.multiple_of(q * 16, 16), 16)],
                            invout.at[pl.ds(base + r0, 16)])
            return 0
        lax.fori_loop(0, mych, out_chunk, 0)
        plsc.subcore_barrier()

    pltpu.sync_copy(
        cntw, counts.at[pl.ds(pl.multiple_of((c * NS + s) * L, 16), L)])


def _replay_body(table, srcc, dlocc, counts, invin, out,
                 srcbuf, dstbuf, sidx_send, dloc_send,
                 sidx2_send, dloc2_send, sidx3_send, dloc3_send,
                 sidx4_send, dloc4_send, rowsbuf, rows2buf, rows3buf, rows4buf,
                 cnt, cntw, acc,
                 sem, sem2, sem3, sem4, ssem, ssem2, ssem3, ssem4):
    c = lax.axis_index("c")
    s = lax.axis_index("s")
    slots = [(sidx_send, dloc_send, rowsbuf, sem, ssem),
             (sidx2_send, dloc2_send, rows2buf, sem2, ssem2),
             (sidx3_send, dloc3_send, rows3buf, sem3, ssem3),
             (sidx4_send, dloc4_send, rows4buf, sem4, ssem4)]
    zero16 = jnp.zeros((L,), jnp.float32)
    iota16 = lax.iota(jnp.int32, L)

    pltpu.sync_copy(
        counts.at[pl.ds(pl.multiple_of((c * NS + s) * L, 16), L)], cntw)

    for b in range(NBPC):
        block = NBPC * c + b
        base = block * DB
        seg = pl.multiple_of((block * NS + s) * SEGCAP, 128)

        _zero_rows64(rowsbuf, zero16)
        _zero_acc(s, acc, rowsbuf)
        plsc.subcore_barrier()

        n_tb = cntw[pl.ds(0, L)][b]
        nbat = (n_tb + RBATCH - 1) // RBATCH

        def stage_r(lofs, gofs, sidx_d, dloc_d):
            fp = n_tb - gofs * RBATCH
            for j in range(RBATCH // L):
                sv = srcbuf[pl.ds(lofs * RBATCH + j * L, L)]
                dv = dstbuf[pl.ds(lofs * RBATCH + j * L, L)]
                lane = j * L + iota16
                dv = jnp.where(lane < fp, dv, DB + (lane & 127))
                sidx_d[pl.ds(j * L, L)] = sv
                dloc_d[pl.ds(j * L, L)] = dv

        def chunk_step(mi, _):
            cofs = pl.multiple_of(seg + mi * MACRO, 8)
            pltpu.sync_copy(srcc.at[pl.ds(cofs, MACRO)], srcbuf)
            pltpu.sync_copy(dlocc.at[pl.ds(cofs, MACRO)], dstbuf)
            nb_c = jnp.minimum(nbat - mi * (MACRO // RBATCH), MACRO // RBATCH)

            def flush_n(lbase, nslots):
                gs = []
                for i in range(nslots):
                    sd, dd, rb, gsm, ssm = slots[i]
                    stage_r(lbase + i, mi * (MACRO // RBATCH) + lbase + i, sd, dd)
                    gs.append(pltpu.async_copy(table.at[sd], rb, gsm))
                ss = []
                for i in range(nslots):
                    sd, dd, rb, gsm, ssm = slots[i]
                    gs[i].wait()
                    ss.append(pltpu.async_copy(rb, acc.at[dd], ssm, add=True))
                for d_ in ss:
                    d_.wait()

            def flush_4(k, _2):
                flush_n(4 * k, 4)
                return 0
            lax.fori_loop(0, nb_c // 4, flush_4, 0)

            @pl.when(nb_c % 4 >= 2)
            def _():
                flush_n(nb_c - (nb_c % 4), 2)

            @pl.when(nb_c % 2 == 1)
            def _():
                flush_n(nb_c - 1, 1)
            return 0
        nch = (nbat + (MACRO // RBATCH) - 1) // (MACRO // RBATCH)
        lax.fori_loop(0, nch, chunk_step, 0)
        plsc.subcore_barrier()

        mych, start_row = _own_rows(s, block)

        def out_chunk(q, _):
            r0 = pl.multiple_of(start_row + q * 16, 16)
            pltpu.sync_copy(acc.at[pl.ds(r0, 16)], rowsbuf.at[pl.ds(0, 16)])
            pltpu.sync_copy(invin.at[pl.ds(base + r0, 16)],
                            cnt.at[pl.ds(pl.multiple_of(q * 16, 16), 16)])
            _scale_rows(q, cnt, rowsbuf)
            pltpu.sync_copy(rowsbuf.at[pl.ds(0, 16)],
                            out.at[pl.ds(base + r0, 16)])
            return 0
        lax.fori_loop(0, mych, out_chunk, 0)
        plsc.subcore_barrier()


_MESH = plsc.VectorSubcoreMesh(core_axis_name="c", subcore_axis_name="s",
                               num_cores=NC, num_subcores=NS)
_SEGTOT = NBLK * NS * SEGCAP


@jax.jit
def _smean_scan(table, srcp, dstp):
    f = pl.kernel(
        _scan_body,
        out_type=(
            jax.ShapeDtypeStruct((NROWS, D), jnp.float32),
            jax.ShapeDtypeStruct((_SEGTOT,), jnp.int32),
            jax.ShapeDtypeStruct((_SEGTOT,), jnp.int32),
            jax.ShapeDtypeStruct((NC * NS * L,), jnp.int32),
            jax.ShapeDtypeStruct((NROWS,), jnp.float32),
        ),
        mesh=_MESH,
        scratch_types=[
            pltpu.VMEM((MACRO,), jnp.int32),
            pltpu.VMEM((MACRO,), jnp.int32),
            pltpu.VMEM((MACRO + SBATCH,), jnp.int32),
            pltpu.VMEM((MACRO + SBATCH,), jnp.int32),
            pltpu.VMEM((SBATCH,), jnp.int32),
            pltpu.VMEM((SBATCH,), jnp.int32),
            pltpu.VMEM((SBATCH,), jnp.int32),
            pltpu.VMEM((SBATCH,), jnp.int32),
            pltpu.VMEM((SBATCH, D), jnp.float32),
            pltpu.VMEM((SBATCH, D), jnp.float32),
            pltpu.VMEM((DB,), jnp.float32),
            pltpu.VMEM((NS * CW,), jnp.float32),
            pltpu.VMEM((L,), jnp.int32),
            pltpu.VMEM_SHARED((ACC_ROWS, D), jnp.float32),
            pltpu.VMEM_SHARED((NS * DB,), jnp.float32),
        ] + [pltpu.SemaphoreType.DMA] * 6,
        compiler_params=pltpu.CompilerParams(needs_layout_passes=False),
    )
    return f(table, srcp, dstp)


@jax.jit
def _smean_replay(table, srcc, dlocc, counts, invin):
    f = pl.kernel(
        _replay_body,
        out_type=jax.ShapeDtypeStruct((NROWS, D), jnp.float32),
        mesh=_MESH,
        scratch_types=[
            pltpu.VMEM((MACRO,), jnp.int32),
            pltpu.VMEM((MACRO,), jnp.int32),
            pltpu.VMEM((RBATCH,), jnp.int32),
            pltpu.VMEM((RBATCH,), jnp.int32),
            pltpu.VMEM((RBATCH,), jnp.int32),
            pltpu.VMEM((RBATCH,), jnp.int32),
            pltpu.VMEM((RBATCH,), jnp.int32),
            pltpu.VMEM((RBATCH,), jnp.int32),
            pltpu.VMEM((RBATCH,), jnp.int32),
            pltpu.VMEM((RBATCH,), jnp.int32),
            pltpu.VMEM((RBATCH, D), jnp.float32),
            pltpu.VMEM((RBATCH, D), jnp.float32),
            pltpu.VMEM((RBATCH, D), jnp.float32),
            pltpu.VMEM((RBATCH, D), jnp.float32),
            pltpu.VMEM((DB,), jnp.float32),
            pltpu.VMEM((L,), jnp.int32),
            pltpu.VMEM_SHARED((ACC_ROWS, D), jnp.float32),
        ] + [pltpu.SemaphoreType.DMA] * 8,
        compiler_params=pltpu.CompilerParams(needs_layout_passes=False),
    )
    return f(table, srcc, dlocc, counts, invin)


def _pad_edges(e):
    src = e[0].astype(jnp.int32)
    dst = e[1].astype(jnp.int32)
    pad = EP - E
    srcp = jnp.concatenate([src, jnp.zeros((pad,), jnp.int32)])
    dstp = jnp.concatenate([dst, jnp.full((pad,), -1, jnp.int32)])
    return srcp, dstp


def kernel(x_users, x_artists, edge_index_a2u, edge_index_u2a):
    sa, da = _pad_edges(edge_index_a2u)
    su, du = _pad_edges(edge_index_u2a)
    xu, xa = x_users, x_artists
    xu, a_srcc, a_dlocc, a_counts, a_inv = _smean_scan(xa, sa, da)
    xa, u_srcc, u_dlocc, u_counts, u_inv = _smean_scan(xu, su, du)
    fu = x_users + xu
    fa = x_artists + xa
    for _ in range(2):
        xu = _smean_replay(xa, a_srcc, a_dlocc, a_counts, a_inv)
        xa = _smean_replay(xu, u_srcc, u_dlocc, u_counts, u_inv)
        fu = fu + xu
        fa = fa + xa
    return (0.25 * fu, 0.25 * fa)
